# Initial kernel scaffold; baseline (speedup 1.0000x reference)
import jax
import jax.numpy as jnp
from jax.experimental import pallas as pl


def kernel(h, adj, W, a):
    raise NotImplementedError("write your pallas kernel here")


# speedup vs baseline: 2.3872x; 2.3872x over previous
import jax
import jax.numpy as jnp
from jax.experimental import pallas as pl

_ALPHA = 0.5
_NEG = -9e15


def _prep_kernel(h_ref, w_ref, wh_ref):
    wh_ref[...] = jnp.dot(h_ref[...], w_ref[...],
                          preferred_element_type=jnp.float32)


def _gat_kernel(adj_ref, wh_ref, a1_ref, a2_ref, out_ref):
    i = pl.program_id(0)
    r = out_ref.shape[0]
    wh = wh_ref[...]
    wh_blk = wh_ref[pl.ds(i * r, r), :]
    wh1 = jax.lax.dot_general(wh_blk, a1_ref[...],
                              (((1,), (1,)), ((), ())),
                              preferred_element_type=jnp.float32)
    wh2t = jax.lax.dot_general(a2_ref[...], wh,
                               (((1,), (1,)), ((), ())),
                               preferred_element_type=jnp.float32)
    e = wh1 + wh2t
    e = jnp.where(e >= 0, e, _ALPHA * e)
    logits = jnp.where(adj_ref[...] > 0, e, _NEG)
    m = jnp.max(logits, axis=1, keepdims=True)
    p = jnp.exp(logits - m)
    att = p / jnp.sum(p, axis=1, keepdims=True)
    hp = jax.lax.dot_general(att, wh, (((1,), (0,)), ((), ())),
                             preferred_element_type=jnp.float32)
    out_ref[...] = jnp.maximum(hp, 0.0)


def kernel(h, adj, W, a):
    n, in_ch = h.shape
    out_ch = W.shape[1]
    a1 = a[:out_ch].reshape(1, out_ch)
    a2 = a[out_ch:].reshape(1, out_ch)

    rp = 2000 if n % 2000 == 0 else n
    wh = pl.pallas_call(
        _prep_kernel,
        grid=(n // rp,),
        in_specs=[pl.BlockSpec((rp, in_ch), lambda i: (i, 0)),
                  pl.BlockSpec((in_ch, out_ch), lambda i: (0, 0))],
        out_specs=pl.BlockSpec((rp, out_ch), lambda i: (i, 0)),
        out_shape=jax.ShapeDtypeStruct((n, out_ch), jnp.float32),
    )(h, W)

    r = 400 if n % 400 == 0 else n
    out = pl.pallas_call(
        _gat_kernel,
        grid=(n // r,),
        in_specs=[pl.BlockSpec((r, n), lambda i: (i, 0)),
                  pl.BlockSpec((n, out_ch), lambda i: (0, 0)),
                  pl.BlockSpec((1, out_ch), lambda i: (0, 0)),
                  pl.BlockSpec((1, out_ch), lambda i: (0, 0))],
        out_specs=pl.BlockSpec((r, out_ch), lambda i: (i, 0)),
        out_shape=jax.ShapeDtypeStruct((n, out_ch), jnp.float32),
    )(adj, wh, a1, a2)
    return out


# speedup vs baseline: 2.8895x; 1.2104x over previous
import jax
import jax.numpy as jnp
from jax.experimental import pallas as pl
from jax.experimental.pallas import tpu as pltpu

_ALPHA = 0.5


def _prep_kernel(h_ref, w_ref, wh_ref):
    wh_ref[...] = jnp.dot(h_ref[...], w_ref[...],
                          preferred_element_type=jnp.float32)


def _gat_kernel(adj_ref, wh_ref, a1_ref, a2_ref, out_ref,
                wh2t_ref, csum_ref, m2_ref):
    i = pl.program_id(0)
    r = out_ref.shape[0]
    n = adj_ref.shape[1]
    wh = wh_ref[...]

    @pl.when(i == 0)
    def _():
        w2 = jax.lax.dot_general(a2_ref[...], wh,
                                 (((1,), (1,)), ((), ())),
                                 preferred_element_type=jnp.float32)
        wh2t_ref[...] = w2
        m2_ref[...] = jnp.max(w2, axis=(0, 1), keepdims=True)
        ones = jnp.ones((1, n), dtype=jnp.float32)
        csum_ref[...] = jax.lax.dot_general(ones, wh,
                                            (((1,), (0,)), ((), ())),
                                            preferred_element_type=jnp.float32)

    wh_blk = wh_ref[pl.ds(i * r, r), :]
    wh1 = jax.lax.dot_general(wh_blk, a1_ref[...],
                              (((1,), (1,)), ((), ())),
                              preferred_element_type=jnp.float32)
    u = wh1 + m2_ref[...]
    mi = jnp.maximum(u, _ALPHA * u)
    t = wh1 + wh2t_ref[...]
    e = jnp.maximum(t, _ALPHA * t)
    p = adj_ref[...] * jnp.exp(e - mi)
    s = jnp.sum(p, axis=1, keepdims=True)
    pm = jax.lax.dot_general(p, wh, (((1,), (0,)), ((), ())),
                             preferred_element_type=jnp.float32)
    safe = jnp.where(s > 0, s, 1.0)
    hp = jnp.where(s > 0, pm / safe, csum_ref[...] / n)
    out_ref[...] = jnp.maximum(hp, 0.0)


def kernel(h, adj, W, a):
    n, in_ch = h.shape
    out_ch = W.shape[1]
    a1 = a[:out_ch].reshape(1, out_ch)
    a2 = a[out_ch:].reshape(1, out_ch)

    rp = 2000 if n % 2000 == 0 else n
    wh = pl.pallas_call(
        _prep_kernel,
        grid=(n // rp,),
        in_specs=[pl.BlockSpec((rp, in_ch), lambda i: (i, 0)),
                  pl.BlockSpec((in_ch, out_ch), lambda i: (0, 0))],
        out_specs=pl.BlockSpec((rp, out_ch), lambda i: (i, 0)),
        out_shape=jax.ShapeDtypeStruct((n, out_ch), jnp.float32),
    )(h, W)

    r = 400 if n % 400 == 0 else n
    out = pl.pallas_call(
        _gat_kernel,
        grid=(n // r,),
        in_specs=[pl.BlockSpec((r, n), lambda i: (i, 0)),
                  pl.BlockSpec((n, out_ch), lambda i: (0, 0)),
                  pl.BlockSpec((1, out_ch), lambda i: (0, 0)),
                  pl.BlockSpec((1, out_ch), lambda i: (0, 0))],
        out_specs=pl.BlockSpec((r, out_ch), lambda i: (i, 0)),
        out_shape=jax.ShapeDtypeStruct((n, out_ch), jnp.float32),
        scratch_shapes=[pltpu.VMEM((1, n), jnp.float32),
                        pltpu.VMEM((1, out_ch), jnp.float32),
                        pltpu.VMEM((1, 1), jnp.float32)],
    )(adj, wh, a1, a2)
    return out


# speedup vs baseline: 3.8277x; 1.3247x over previous
import jax
import jax.numpy as jnp
from jax.experimental import pallas as pl
from jax.experimental.pallas import tpu as pltpu

_ALPHA = 0.5
_LOG2E = 1.4426950408889634


def _prep_kernel(h_ref, w_ref, wh_ref):
    wh_ref[...] = jnp.dot(h_ref[...], w_ref[...],
                          preferred_element_type=jnp.float32)


def _gat_kernel(adj_ref, wh_ref, whaug_ref, a1_ref, a2_ref, out_ref,
                w2c_ref, w2ca_ref, csum_ref, m2_ref):
    i = pl.program_id(0)
    r = out_ref.shape[0]
    n = adj_ref.shape[1]
    c = out_ref.shape[1]

    @pl.when(i == 0)
    def _():
        w2 = jax.lax.dot_general(a2_ref[...], wh_ref[...],
                                 (((1,), (1,)), ((), ())),
                                 preferred_element_type=jnp.float32)
        m2_ref[...] = jnp.max(w2, axis=(0, 1), keepdims=True)
        w2c_ref[...] = w2 * _LOG2E
        w2ca_ref[...] = w2 * (_ALPHA * _LOG2E)
        ones = jnp.ones((1, n), dtype=jnp.float32)
        csum_ref[...] = jax.lax.dot_general(ones, wh_ref[...],
                                            (((1,), (0,)), ((), ())),
                                            preferred_element_type=jnp.float32)

    wh_blk = wh_ref[pl.ds(i * r, r), :]
    wh1 = jax.lax.dot_general(wh_blk, a1_ref[...],
                              (((1,), (1,)), ((), ())),
                              preferred_element_type=jnp.float32)
    u = wh1 + m2_ref[...]
    mi = jnp.maximum(u, _ALPHA * u)
    q1 = _LOG2E * (wh1 - mi)
    q2 = _LOG2E * (_ALPHA * wh1 - mi)
    arg = jnp.maximum(q1 + w2c_ref[...], q2 + w2ca_ref[...])
    p = adj_ref[...] * jnp.exp2(arg)
    pb = p.astype(jnp.bfloat16)
    pm = jax.lax.dot_general(pb, whaug_ref[...], (((1,), (0,)), ((), ())),
                             preferred_element_type=jnp.float32)
    s = pm[:, c:c + 1]
    safe = jnp.where(s > 0, s, 1.0)
    hp = jnp.where(s > 0, pm[:, :c] / safe, csum_ref[...] / n)
    out_ref[...] = jnp.maximum(hp, 0.0)


def kernel(h, adj, W, a):
    n, in_ch = h.shape
    out_ch = W.shape[1]
    a1 = a[:out_ch].reshape(1, out_ch)
    a2 = a[out_ch:].reshape(1, out_ch)

    rp = 2000 if n % 2000 == 0 else n
    wh = pl.pallas_call(
        _prep_kernel,
        grid=(n // rp,),
        in_specs=[pl.BlockSpec((rp, in_ch), lambda i: (i, 0)),
                  pl.BlockSpec((in_ch, out_ch), lambda i: (0, 0))],
        out_specs=pl.BlockSpec((rp, out_ch), lambda i: (i, 0)),
        out_shape=jax.ShapeDtypeStruct((n, out_ch), jnp.float32),
    )(h, W)

    whaug = jnp.concatenate(
        [wh.astype(jnp.bfloat16),
         jnp.ones((n, 1), dtype=jnp.bfloat16),
         jnp.zeros((n, out_ch - 1), dtype=jnp.bfloat16)], axis=1)

    r = 400 if n % 400 == 0 else n
    out = pl.pallas_call(
        _gat_kernel,
        grid=(n // r,),
        in_specs=[pl.BlockSpec((r, n), lambda i: (i, 0)),
                  pl.BlockSpec((n, out_ch), lambda i: (0, 0)),
                  pl.BlockSpec((n, 2 * out_ch), lambda i: (0, 0)),
                  pl.BlockSpec((1, out_ch), lambda i: (0, 0)),
                  pl.BlockSpec((1, out_ch), lambda i: (0, 0))],
        out_specs=pl.BlockSpec((r, out_ch), lambda i: (i, 0)),
        out_shape=jax.ShapeDtypeStruct((n, out_ch), jnp.float32),
        scratch_shapes=[pltpu.VMEM((1, n), jnp.float32),
                        pltpu.VMEM((1, n), jnp.float32),
                        pltpu.VMEM((1, out_ch), jnp.float32),
                        pltpu.VMEM((1, 1), jnp.float32)],
    )(adj, wh, whaug, a1, a2)
    return out
